# baseline (speedup 1.0000x reference)
import functools

import jax
import jax.numpy as jnp
from jax import lax
from jax.experimental import pallas as pl
from jax.experimental.pallas import tpu as pltpu
from jax.experimental.pallas import tpu_sc as plsc


def kernel(images, labels):
    B, H, W, C = images.shape
    n_workers = 32
    rows = H // n_workers

    @functools.partial(
        pl.kernel,
        out_type=[
            jax.ShapeDtypeStruct(images.shape, images.dtype),
            jax.ShapeDtypeStruct(labels.shape, labels.dtype),
        ],
        mesh=plsc.VectorSubcoreMesh(core_axis_name="c", subcore_axis_name="s"),
    )
    def _copy(img_hbm, lab_hbm, img_out, lab_out):
        c = lax.axis_index("c")
        s = lax.axis_index("s")
        wid = s * 2 + c
        base = wid * rows
        pltpu.sync_copy(
            img_hbm.at[:, pl.ds(base, rows)],
            img_out.at[:, pl.ds(base, rows)],
        )

        @pl.when(jnp.logical_and(c == 0, s == 0))
        def _():
            pltpu.sync_copy(lab_hbm, lab_out)

    return tuple(_copy(images, labels))


# speedup vs baseline: 3.7762x; 3.7762x over previous
import functools

import jax
import jax.numpy as jnp
from jax import lax
from jax.experimental import pallas as pl
from jax.experimental.pallas import tpu as pltpu
from jax.experimental.pallas import tpu_sc as plsc


def kernel(images, labels):
    B, H, W, C = images.shape
    n_img = B * H * W * C
    img_flat = images.reshape(n_img)
    lab_flat = labels.reshape(labels.size)
    n_workers = 32
    chunk = n_img // n_workers

    @functools.partial(
        pl.kernel,
        out_type=[
            jax.ShapeDtypeStruct(img_flat.shape, img_flat.dtype),
            jax.ShapeDtypeStruct(lab_flat.shape, lab_flat.dtype),
        ],
        mesh=plsc.VectorSubcoreMesh(core_axis_name="c", subcore_axis_name="s"),
    )
    def _copy(img_hbm, lab_hbm, img_out, lab_out):
        c = lax.axis_index("c")
        s = lax.axis_index("s")
        wid = s * 2 + c
        base = wid * chunk
        pltpu.sync_copy(
            img_hbm.at[pl.ds(base, chunk)],
            img_out.at[pl.ds(base, chunk)],
        )

        @pl.when(jnp.logical_and(c == 0, s == 0))
        def _():
            pltpu.sync_copy(lab_hbm, lab_out)

    out_img, out_lab = _copy(img_flat, lab_flat)
    return (out_img.reshape(B, H, W, C), out_lab.reshape(labels.shape))


# speedup vs baseline: 1667.5822x; 441.5984x over previous
import jax
import jax.numpy as jnp
from jax.experimental import pallas as pl
from jax.experimental.pallas import tpu as pltpu


def _passthrough(img_ref, lab_ref, img_out, lab_out):
    img_out[...] = img_ref[...]
    lab_out[...] = lab_ref[...]


def kernel(images, labels):
    B, H, W, C = images.shape
    img2 = images.transpose(0, 3, 1, 2).reshape(B * C * H, W)
    rows, cols = img2.shape
    grid = 16
    blk = rows // grid
    out_img, out_lab = pl.pallas_call(
        _passthrough,
        grid=(grid,),
        in_specs=[
            pl.BlockSpec((blk, cols), lambda i: (i, 0)),
            pl.BlockSpec(labels.shape, lambda i: (0, 0)),
        ],
        out_specs=[
            pl.BlockSpec((blk, cols), lambda i: (i, 0)),
            pl.BlockSpec(labels.shape, lambda i: (0, 0)),
        ],
        out_shape=[
            jax.ShapeDtypeStruct(img2.shape, img2.dtype),
            jax.ShapeDtypeStruct(labels.shape, labels.dtype),
        ],
    )(img2, labels)
    return (out_img.reshape(B, C, H, W).transpose(0, 2, 3, 1), out_lab)
